# baseline (speedup 1.0000x reference)
import functools

import jax
import jax.numpy as jnp
from jax import lax
from jax.experimental import pallas as pl
from jax.experimental.pallas import tpu as pltpu
from jax.experimental.pallas import tpu_sc as plsc

N = 1000000
DIM = 64
NPAIR = N // 2


def _diag_consts():
    iota = lax.iota(jnp.int32, 16)
    return iota, [(iota + j) & 15 for j in range(16)]


def _transpose_block(blk, tp, ncols):
    iota = lax.iota(jnp.int32, 16)
    d0iota = [iota + 16 * kd for kd in range(4)]

    @plsc.parallel_loop(0, ncols, unroll=4)
    def _(i):
        cols = ((i >> 4) << 4) + ((iota + i) & 15)
        qv = cols >> 1
        hv = (cols & 1) << 6
        for kd in range(4):
            vals = plsc.load_gather(blk, [d0iota[kd], cols])
            plsc.store_scatter(tp, [qv, hv + d0iota[kd]], vals)


def kernel(idx, embed_params):
    B, F = idx.shape
    tabT = embed_params.T
    idxT = idx.T.astype(jnp.int32)

    info = plsc.get_sparse_core_info()
    nc, ns = info.num_cores, info.num_subcores
    nw = nc * ns
    CB = 256
    nblk_full = N // CB
    tail_cols = N - nblk_full * CB
    per_w = (nblk_full + nw - 1) // nw
    b_per_w = B // nw
    RB = 256
    NBC = b_per_w // RB

    mesh = plsc.VectorSubcoreMesh(core_axis_name="c", subcore_axis_name="s")
    params = pltpu.CompilerParams(use_tc_tiling_on_sc=True,
                                  needs_layout_passes=False,
                                  disable_bounds_checks=True)

    @functools.partial(
        pl.kernel,
        mesh=mesh,
        out_type=jax.ShapeDtypeStruct((NPAIR, 128), jnp.float32),
        scratch_types=(
            [pltpu.VMEM((64, CB), jnp.float32)] * 3
            + [pltpu.VMEM((CB // 2, 128), jnp.float32)] * 3
            + [pltpu.SemaphoreType.DMA] * 6
        ),
        compiler_params=params,
    )
    def ka(tabT_hbm, tail_hbm, scr_hbm,
           blk0, blk1, blk2, tp0, tp1, tp2,
           is0, is1, is2, os0, os1, os2):
        wid = lax.axis_index("s") * nc + lax.axis_index("c")
        base = wid * per_w
        lim = jnp.minimum(base + per_w, nblk_full)
        blks = (blk0, blk1, blk2)
        tps = (tp0, tp1, tp2)
        isems = (is0, is1, is2)
        osems = (os0, os1, os2)

        def in_start(e, par):
            pltpu.async_copy(tabT_hbm.at[:, pl.ds(CB * e, CB)],
                             blks[par], isems[par])

        def in_wait(par):
            pltpu.make_async_copy(tabT_hbm.at[:, pl.ds(0, CB)],
                                  blks[par], isems[par]).wait()

        def out_start(e, par):
            pltpu.async_copy(tps[par],
                             scr_hbm.at[pl.ds((CB // 2) * e, CB // 2), :],
                             osems[par])

        def out_wait(par):
            pltpu.make_async_copy(tps[par],
                                  scr_hbm.at[pl.ds(0, CB // 2), :],
                                  osems[par]).wait()

        for k in range(3):
            @pl.when(base + k < lim)
            def _():
                in_start(base + k, k)

        def body(i, carry):
            for par in range(3):
                e = base + 3 * i + par

                @pl.when(e < lim)
                def _():
                    in_wait(par)

                    @pl.when(e - base >= 3)
                    def _():
                        out_wait(par)

                    _transpose_block(blks[par], tps[par], CB)
                    out_start(e, par)

                    @pl.when(e + 3 < lim)
                    def _():
                        in_start(e + 3, par)

            return carry

        lax.fori_loop(0, (per_w + 2) // 3, body, 0)
        for k in range(3):
            out_wait(k)

        @pl.when(wid == nw - 1)
        def _():
            pltpu.sync_copy(tail_hbm, blk0)
            _transpose_block(blk0, tp0, tail_cols)
            pltpu.sync_copy(
                tp0.at[pl.ds(0, tail_cols // 2), :],
                scr_hbm.at[pl.ds((CB // 2) * nblk_full, tail_cols // 2), :])

    @functools.partial(
        pl.kernel,
        mesh=mesh,
        out_type=jax.ShapeDtypeStruct((F, DIM, B), jnp.float32),
        scratch_types=[
            pltpu.VMEM((F, b_per_w), jnp.int32),
            pltpu.VMEM((RB, 128), jnp.float32),
            pltpu.VMEM((RB, 128), jnp.float32),
            pltpu.VMEM((DIM, RB), jnp.float32),
            pltpu.VMEM((DIM, RB), jnp.float32),
            pltpu.VMEM((RB,), jnp.int32),
            pltpu.VMEM((RB,), jnp.int32),
            pltpu.VMEM((RB,), jnp.int32),
            pltpu.VMEM((RB,), jnp.int32),
            pltpu.SemaphoreType.DMA,
            pltpu.SemaphoreType.DMA,
            pltpu.SemaphoreType.DMA,
            pltpu.SemaphoreType.DMA,
        ],
        compiler_params=params,
    )
    def kb(scr_hbm, idxT_hbm, out_hbm, idx_v, rows0, rows1, ob0, ob1,
           pl0, pl1, hl0, hl1, gs0, gs1, ws0, ws1):
        wid = lax.axis_index("s") * nc + lax.axis_index("c")
        bbase = wid * b_per_w
        rows = (rows0, rows1)
        obufs = (ob0, ob1)
        plists = (pl0, pl1)
        hlists = (hl0, hl1)
        gsems = (gs0, gs1)
        wsems = (ws0, ws1)

        pltpu.sync_copy(idxT_hbm.at[:, pl.ds(bbase, b_per_w)], idx_v)

        def build_ph(f, bc, x):
            for k in range(RB // 16):
                raw = idx_v[f, pl.ds(RB * bc + 16 * k, 16)]
                plists[x][pl.ds(16 * k, 16)] = raw >> 1
                hlists[x][pl.ds(16 * k, 16)] = (raw & 1) << 6

        def gather_start(x):
            h = RB // 2
            pltpu.async_copy(scr_hbm.at[plists[x].at[pl.ds(0, h)]],
                             rows[x].at[pl.ds(0, h), :], gsems[x])
            pltpu.async_copy(scr_hbm.at[plists[x].at[pl.ds(h, h)]],
                             rows[x].at[pl.ds(h, h), :], gsems[x])

        def gather_wait(x):
            h = RB // 2
            for o in (0, h):
                pltpu.make_async_copy(scr_hbm.at[plists[x].at[pl.ds(o, h)]],
                                      rows[x].at[pl.ds(o, h), :],
                                      gsems[x]).wait()

        def write_start(f, bc, x):
            pltpu.async_copy(obufs[x],
                             out_hbm.at[f, :, pl.ds(bbase + RB * bc, RB)],
                             wsems[x])

        def write_wait(x):
            pltpu.make_async_copy(obufs[x],
                                  out_hbm.at[0, :, pl.ds(0, RB)],
                                  wsems[x]).wait()

        iota, cdiag = _diag_consts()

        def vpu_round(x):
            @plsc.parallel_loop(0, RB // 16, unroll=2)
            def _(m):
                brows = jnp.full((16,), 0, jnp.int32) + m * 16 + iota
                bcols = jnp.full((16,), 0, jnp.int32) + m * 16 + iota
                hvec = hlists[x][pl.ds(16 * m, 16)]
                for kd in range(4):
                    for j in range(16):
                        dcr = cdiag[j] + 16 * kd
                        vals = plsc.load_gather(rows[x], [brows, hvec + dcr])
                        plsc.store_scatter(obufs[x], [dcr, bcols], vals)

        build_ph(0, 0, 0)
        gather_start(0)
        TOT = NBC * F

        def pair_body(i, carry):
            bc, f = carry
            for fh in range(2):
                rg = 2 * i + fh
                gather_wait(fh)
                wrap = f + 1 == F
                nf = jnp.where(wrap, 0, f + 1)
                nbc = bc + wrap.astype(jnp.int32)

                @pl.when(rg + 1 < TOT)
                def _():
                    build_ph(nf, nbc, 1 - fh)
                    gather_start(1 - fh)

                @pl.when(rg >= 2)
                def _():
                    write_wait(fh)

                vpu_round(fh)
                write_start(f, bc, fh)
                bc, f = nbc, nf
            return bc, f

        lax.fori_loop(0, TOT // 2, pair_body,
                      (jnp.int32(0), jnp.int32(0)))
        write_wait(0)
        write_wait(1)

    tailT = jnp.pad(embed_params[N - tail_cols:].T,
                    ((0, 0), (0, CB - tail_cols)))
    scr = ka(tabT, tailT)
    outT = kb(scr, idxT)
    return outT.transpose(2, 0, 1)


# speedup vs baseline: 1.0116x; 1.0116x over previous
import functools

import jax
import jax.numpy as jnp
from jax import lax
from jax.experimental import pallas as pl
from jax.experimental.pallas import tpu as pltpu
from jax.experimental.pallas import tpu_sc as plsc

N = 1000000
DIM = 64
NPAIR = N // 2


def _diag_consts():
    iota = lax.iota(jnp.int32, 16)
    return iota, [(iota + j) & 15 for j in range(16)]


def _transpose_block(blk, tp, ncols):
    iota = lax.iota(jnp.int32, 16)
    d0iota = [iota + 16 * kd for kd in range(4)]

    @plsc.parallel_loop(0, ncols, unroll=4)
    def _(i):
        cols = ((i >> 4) << 4) + ((iota + i) & 15)
        qv = cols >> 1
        hv = (cols & 1) << 6
        for kd in range(4):
            vals = plsc.load_gather(blk, [d0iota[kd], cols])
            plsc.store_scatter(tp, [qv, hv + d0iota[kd]], vals)


def kernel(idx, embed_params):
    B, F = idx.shape
    tabT = embed_params.T
    idxT = idx.T.astype(jnp.int32)

    info = plsc.get_sparse_core_info()
    nc, ns = info.num_cores, info.num_subcores
    nw = nc * ns
    CB = 256
    nblk_full = N // CB
    tail_cols = N - nblk_full * CB
    per_w = (nblk_full + nw - 1) // nw
    b_per_w = B // nw
    RB = 256
    NBC = b_per_w // RB

    mesh = plsc.VectorSubcoreMesh(core_axis_name="c", subcore_axis_name="s")
    params = pltpu.CompilerParams(use_tc_tiling_on_sc=True,
                                  needs_layout_passes=False)

    @functools.partial(
        pl.kernel,
        mesh=mesh,
        out_type=jax.ShapeDtypeStruct((NPAIR, 128), jnp.float32),
        scratch_types=(
            [pltpu.VMEM((64, CB), jnp.float32)] * 3
            + [pltpu.VMEM((CB // 2, 128), jnp.float32)] * 3
            + [pltpu.SemaphoreType.DMA] * 6
        ),
        compiler_params=params,
    )
    def ka(tabT_hbm, tail_hbm, scr_hbm,
           blk0, blk1, blk2, tp0, tp1, tp2,
           is0, is1, is2, os0, os1, os2):
        wid = lax.axis_index("s") * nc + lax.axis_index("c")
        base = wid * per_w
        lim = jnp.minimum(base + per_w, nblk_full)
        blks = (blk0, blk1, blk2)
        tps = (tp0, tp1, tp2)
        isems = (is0, is1, is2)
        osems = (os0, os1, os2)

        def in_start(e, par):
            pltpu.async_copy(tabT_hbm.at[:, pl.ds(CB * e, CB)],
                             blks[par], isems[par])

        def in_wait(par):
            pltpu.make_async_copy(tabT_hbm.at[:, pl.ds(0, CB)],
                                  blks[par], isems[par]).wait()

        def out_start(e, par):
            pltpu.async_copy(tps[par],
                             scr_hbm.at[pl.ds((CB // 2) * e, CB // 2), :],
                             osems[par])

        def out_wait(par):
            pltpu.make_async_copy(tps[par],
                                  scr_hbm.at[pl.ds(0, CB // 2), :],
                                  osems[par]).wait()

        for k in range(3):
            @pl.when(base + k < lim)
            def _():
                in_start(base + k, k)

        def body(i, carry):
            for par in range(3):
                e = base + 3 * i + par

                @pl.when(e < lim)
                def _():
                    in_wait(par)

                    @pl.when(e - base >= 3)
                    def _():
                        out_wait(par)

                    _transpose_block(blks[par], tps[par], CB)
                    out_start(e, par)

                    @pl.when(e + 3 < lim)
                    def _():
                        in_start(e + 3, par)

            return carry

        lax.fori_loop(0, (per_w + 2) // 3, body, 0)
        for k in range(3):
            out_wait(k)

        @pl.when(wid == nw - 1)
        def _():
            pltpu.sync_copy(tail_hbm, blk0)
            _transpose_block(blk0, tp0, tail_cols)
            pltpu.sync_copy(
                tp0.at[pl.ds(0, tail_cols // 2), :],
                scr_hbm.at[pl.ds((CB // 2) * nblk_full, tail_cols // 2), :])

    @functools.partial(
        pl.kernel,
        mesh=mesh,
        out_type=jax.ShapeDtypeStruct((F, DIM, B), jnp.float32),
        scratch_types=[
            pltpu.VMEM((F, b_per_w), jnp.int32),
            pltpu.VMEM((RB, 128), jnp.float32),
            pltpu.VMEM((RB, 128), jnp.float32),
            pltpu.VMEM((DIM, RB), jnp.float32),
            pltpu.VMEM((DIM, RB), jnp.float32),
            pltpu.VMEM((RB,), jnp.int32),
            pltpu.VMEM((RB,), jnp.int32),
            pltpu.VMEM((RB,), jnp.int32),
            pltpu.VMEM((RB,), jnp.int32),
            pltpu.SemaphoreType.DMA,
            pltpu.SemaphoreType.DMA,
            pltpu.SemaphoreType.DMA,
            pltpu.SemaphoreType.DMA,
        ],
        compiler_params=params,
    )
    def kb(scr_hbm, idxT_hbm, out_hbm, idx_v, rows0, rows1, ob0, ob1,
           pl0, pl1, hl0, hl1, gs0, gs1, ws0, ws1):
        wid = lax.axis_index("s") * nc + lax.axis_index("c")
        bbase = wid * b_per_w
        rows = (rows0, rows1)
        obufs = (ob0, ob1)
        plists = (pl0, pl1)
        hlists = (hl0, hl1)
        gsems = (gs0, gs1)
        wsems = (ws0, ws1)

        pltpu.sync_copy(idxT_hbm.at[:, pl.ds(bbase, b_per_w)], idx_v)

        def build_ph(f, bc, x):
            for k in range(RB // 16):
                raw = idx_v[f, pl.ds(RB * bc + 16 * k, 16)]
                plists[x][pl.ds(16 * k, 16)] = raw >> 1
                hlists[x][pl.ds(16 * k, 16)] = (raw & 1) << 6

        def gather_start(x):
            h = RB // 2
            pltpu.async_copy(scr_hbm.at[plists[x].at[pl.ds(0, h)]],
                             rows[x].at[pl.ds(0, h), :], gsems[x])
            pltpu.async_copy(scr_hbm.at[plists[x].at[pl.ds(h, h)]],
                             rows[x].at[pl.ds(h, h), :], gsems[x])

        def gather_wait(x):
            h = RB // 2
            for o in (0, h):
                pltpu.make_async_copy(scr_hbm.at[plists[x].at[pl.ds(o, h)]],
                                      rows[x].at[pl.ds(o, h), :],
                                      gsems[x]).wait()

        def write_start(f, bc, x):
            pltpu.async_copy(obufs[x],
                             out_hbm.at[f, :, pl.ds(bbase + RB * bc, RB)],
                             wsems[x])

        def write_wait(x):
            pltpu.make_async_copy(obufs[x],
                                  out_hbm.at[0, :, pl.ds(0, RB)],
                                  wsems[x]).wait()

        iota, cdiag = _diag_consts()

        def vpu_round(x):
            @plsc.parallel_loop(0, RB // 16, unroll=2)
            def _(m):
                brows = jnp.full((16,), 0, jnp.int32) + m * 16 + iota
                bcols = jnp.full((16,), 0, jnp.int32) + m * 16 + iota
                hvec = hlists[x][pl.ds(16 * m, 16)]
                for kd in range(4):
                    for j in range(16):
                        dcr = cdiag[j] + 16 * kd
                        vals = plsc.load_gather(rows[x], [brows, hvec + dcr])
                        plsc.store_scatter(obufs[x], [dcr, bcols], vals)

        build_ph(0, 0, 0)
        gather_start(0)
        TOT = NBC * F

        def pair_body(i, carry):
            bc, f = carry
            for fh in range(2):
                rg = 2 * i + fh
                wrap = f + 1 == F
                nf = jnp.where(wrap, 0, f + 1)
                nbc = bc + wrap.astype(jnp.int32)

                @pl.when(rg + 1 < TOT)
                def _():
                    build_ph(nf, nbc, 1 - fh)
                    gather_start(1 - fh)

                gather_wait(fh)

                @pl.when(rg >= 2)
                def _():
                    write_wait(fh)

                vpu_round(fh)
                write_start(f, bc, fh)
                bc, f = nbc, nf
            return bc, f

        lax.fori_loop(0, TOT // 2, pair_body,
                      (jnp.int32(0), jnp.int32(0)))
        write_wait(0)
        write_wait(1)

    tailT = jnp.pad(embed_params[N - tail_cols:].T,
                    ((0, 0), (0, CB - tail_cols)))
    scr = ka(tabT, tailT)
    outT = kb(scr, idxT)
    return outT.transpose(2, 0, 1)
